# baseline (speedup 1.0000x reference)
import functools

import jax
import jax.numpy as jnp
from jax import lax
from jax.experimental import pallas as pl
from jax.experimental.pallas import tpu as pltpu
from jax.experimental.pallas import tpu_sc as plsc

GRID = 64
EPS = 1e-5
C = 128
P = 102400
NW = 32
RPW = P // NW
CHUNK = 128
NCH = RPW // CHUNK
NOFF = 7
BR = 512


def _flat(c):
    return (c[:, 0] * GRID + c[:, 1]) * GRID + c[:, 2]


def _sc_gather(x_pad, idx_ch, noff):
    mesh = plsc.VectorSubcoreMesh(core_axis_name="c", subcore_axis_name="s")

    @functools.partial(
        pl.kernel,
        mesh=mesh,
        out_type=jax.ShapeDtypeStruct((noff * P, C), jnp.float32),
        scratch_types=(
            [pltpu.VMEM((8, CHUNK), jnp.int32)]
            + [pltpu.VMEM((CHUNK, C), jnp.float32) for _ in range(noff)]
            + [pltpu.SemaphoreType.DMA for _ in range(2 * noff)]
        ),
    )
    def gather_kernel(x_hbm, idx_hbm, g_hbm, *scr):
        idx_v = scr[0]
        bufs = scr[1:1 + noff]
        gsems = scr[1 + noff:1 + 2 * noff]
        wsems = scr[1 + 2 * noff:1 + 3 * noff]
        wid = lax.axis_index("s") * 2 + lax.axis_index("c")
        base = wid * RPW

        def chunk_body(ch, carry):
            r0 = base + ch * CHUNK
            pltpu.sync_copy(
                idx_hbm.at[pl.ds((wid * NCH + ch) * 8, 8)], idx_v)
            gh = [
                pltpu.async_copy(x_hbm.at[idx_v.at[o]], bufs[o], gsems[o])
                for o in range(noff)
            ]
            wh = []
            for o in range(noff):
                gh[o].wait()
                wh.append(pltpu.async_copy(
                    bufs[o], g_hbm.at[pl.ds(o * P + r0, CHUNK)], wsems[o]))
            for o in range(noff):
                wh[o].wait()
            return carry

        lax.fori_loop(0, NCH, chunk_body, 0)

    return gather_kernel(x_pad, idx_ch)


def _mm_body(gm_ref, gc_ref, gp_ref, w_ref, f_ref, stats_ref, acc_ref):
    i = pl.program_id(0)

    @pl.when(i == 0)
    def _init():
        acc_ref[...] = jnp.zeros_like(acc_ref)

    f = jnp.zeros((BR, C), jnp.float32)
    for k, g in enumerate((gm_ref, gc_ref, gp_ref)):
        f = f + lax.dot_general(
            g[0], w_ref[k],
            (((1,), (0,)), ((), ())),
            preferred_element_type=jnp.float32,
        )
    f_ref[...] = f.astype(jnp.bfloat16)
    acc_ref[0] += jnp.sum(f, axis=0)
    acc_ref[1] += jnp.sum(f * f, axis=0)
    stats_ref[...] = acc_ref[...]


def _tc_matmul_stats(gm, gc, gp, w):
    grid = (P // BR,)
    (gma, gmo), (gca, gco), (gpa, gpo) = gm, gc, gp

    def spec(off):
        return pl.BlockSpec((1, BR, C), lambda i, o=off: (o, i, 0))

    return pl.pallas_call(
        _mm_body,
        grid=grid,
        in_specs=[
            spec(gmo), spec(gco), spec(gpo),
            pl.BlockSpec((3, C, C), lambda i: (0, 0, 0)),
        ],
        out_specs=[
            pl.BlockSpec((BR, C), lambda i: (i, 0)),
            pl.BlockSpec((2, C), lambda i: (0, 0)),
        ],
        out_shape=[
            jax.ShapeDtypeStruct((P, C), jnp.bfloat16),
            jax.ShapeDtypeStruct((2, C), jnp.float32),
        ],
        scratch_shapes=[pltpu.VMEM((2, C), jnp.float32)],
    )(gma.reshape(-1, P, C), gca.reshape(-1, P, C), gpa.reshape(-1, P, C), w)


def _finalize_body(f1_ref, f2_ref, f3_ref, stats_ref, x_ref, n_inv_ref,
                   o_ref):
    n_inv = n_inv_ref[0]
    s = jnp.zeros_like(x_ref)
    for d, f_ref in enumerate((f1_ref, f2_ref, f3_ref)):
        m = stats_ref[2 * d] * n_inv
        var = stats_ref[2 * d + 1] * n_inv - m * m
        inv = 1.0 / jnp.sqrt(var + EPS)
        fd = f_ref[...].astype(jnp.float32)
        s = s + jax.nn.sigmoid((fd - m[None, :]) * inv[None, :])
    o_ref[...] = s * x_ref[...]


def _tc_finalize(f1, f2, f3, stats, x_pad, n):
    n_inv = jnp.full((1,), 1.0 / n, jnp.float32)
    grid = (P // BR,)
    return pl.pallas_call(
        _finalize_body,
        grid=grid,
        in_specs=[
            pl.BlockSpec((BR, C), lambda i: (i, 0)),
            pl.BlockSpec((BR, C), lambda i: (i, 0)),
            pl.BlockSpec((BR, C), lambda i: (i, 0)),
            pl.BlockSpec((6, C), lambda i: (0, 0)),
            pl.BlockSpec((BR, C), lambda i: (i, 0)),
            pl.BlockSpec(memory_space=pltpu.SMEM),
        ],
        out_specs=pl.BlockSpec((BR, C), lambda i: (i, 0)),
        out_shape=jax.ShapeDtypeStruct((P, C), jnp.float32),
    )(f1, f2, f3, stats, x_pad, n_inv)


def _chunked_idx(idx_rows):
    noff = idx_rows.shape[0]
    t = idx_rows.reshape(noff, NW, NCH, CHUNK).transpose(1, 2, 0, 3)
    t = jnp.concatenate(
        [t, jnp.zeros((NW, NCH, 8 - noff, CHUNK), jnp.int32)], axis=2)
    return t.reshape(NW * NCH * 8, CHUNK)


def kernel(voxel_features, coors, Wa, Wb, Wc):
    n = voxel_features.shape[0]
    grid = jnp.full((GRID * GRID * GRID,), -1, dtype=jnp.int32)
    grid = grid.at[_flat(coors)].max(jnp.arange(n, dtype=jnp.int32))

    sentinel = n + (jnp.arange(n, dtype=jnp.int32) % (P - n))
    pad_sent = n + jnp.arange(P - n, dtype=jnp.int32)
    offs = ((-1, 0, 0), (1, 0, 0), (0, -1, 0), (0, 1, 0), (0, 0, -1),
            (0, 0, 1), (0, 0, 0))
    idx_list = []
    for (dx, dy, dz) in offs:
        nb = coors + jnp.array([dx, dy, dz], coors.dtype)
        valid = jnp.all((nb >= 0) & (nb < GRID), axis=1)
        fl = jnp.where(valid, _flat(nb), 0)
        ii = grid[fl]
        valid = valid & (ii >= 0)
        idx_list.append(jnp.where(valid, ii, sentinel))
    idx = jnp.stack(idx_list)
    idx = jnp.concatenate(
        [idx, jnp.broadcast_to(pad_sent, (NOFF, P - n))], axis=1)

    x_pad = jnp.pad(voxel_features, ((0, P - n), (0, 0)))

    g1 = _sc_gather(x_pad, _chunked_idx(idx[jnp.array([0, 1, 6])]), 3)
    g2 = _sc_gather(x_pad, _chunked_idx(idx[jnp.array([2, 3])]), 2)
    g3 = _sc_gather(x_pad, _chunked_idx(idx[jnp.array([4, 5])]), 2)

    f1, st1 = _tc_matmul_stats((g1, 0), (g1, 2), (g1, 1), Wa)
    f2, st2 = _tc_matmul_stats((g2, 0), (g1, 2), (g2, 1), Wb)
    f3, st3 = _tc_matmul_stats((g3, 0), (g1, 2), (g3, 1), Wc)

    stats = jnp.concatenate([st1, st2, st3], axis=0)
    out = _tc_finalize(f1, f2, f3, stats, x_pad, n)
    return out[:n]
